# baseline (speedup 1.0000x reference)
import jax
import jax.numpy as jnp
from jax import lax
from jax.experimental import pallas as pl
from jax.experimental.pallas import tpu as pltpu
from jax.experimental.pallas import tpu_sc as plsc

_NC = 2
_NS = 16
_NW = _NC * _NS

_TOTAL_ROWS = 2 * 4096
_ROWS_PER_W = _TOTAL_ROWS // _NW
_SRC_OFFSET = 1 * 4096


_K = 8
_ROWS_PER_DMA = _ROWS_PER_W // _K


def _copy_body(x_hbm, out_hbm, sem):
    wid = lax.axis_index("s") * _NC + lax.axis_index("c")
    base = wid * _ROWS_PER_W
    copies = []
    for j in range(_K):
        off = base + j * _ROWS_PER_DMA
        c = pltpu.make_async_copy(
            x_hbm.at[pl.ds(_SRC_OFFSET + off, _ROWS_PER_DMA)],
            out_hbm.at[pl.ds(off, _ROWS_PER_DMA)],
            sem,
        )
        c.start()
        copies.append(c)
    for c in copies:
        c.wait()


def kernel(x):
    x2 = x.reshape(4 * 4096, 4096)
    mesh = plsc.VectorSubcoreMesh(core_axis_name="c", subcore_axis_name="s")
    out = pl.kernel(
        _copy_body,
        mesh=mesh,
        out_type=jax.ShapeDtypeStruct((_TOTAL_ROWS, 4096), jnp.float32),
        scratch_types=[pltpu.SemaphoreType.DMA],
    )(x2)
    return out.reshape(2, 4096, 4096)


# speedup vs baseline: 1.0047x; 1.0047x over previous
import jax
import jax.numpy as jnp
from jax.experimental import pallas as pl
from jax.experimental.pallas import tpu as pltpu

_TOTAL_ROWS = 2 * 4096
_SRC_OFFSET = 1 * 4096
_K = 8
_ROWS_PER_DMA = _TOTAL_ROWS // _K


def _copy_body(x_hbm, out_hbm, sems):
    copies = []
    for j in range(_K):
        off = j * _ROWS_PER_DMA
        c = pltpu.make_async_copy(
            x_hbm.at[pl.ds(_SRC_OFFSET + off, _ROWS_PER_DMA)],
            out_hbm.at[pl.ds(off, _ROWS_PER_DMA)],
            sems.at[j],
        )
        c.start()
        copies.append(c)
    for c in copies:
        c.wait()


def kernel(x):
    x2 = x.reshape(4 * 4096, 4096)
    out = pl.pallas_call(
        _copy_body,
        in_specs=[pl.BlockSpec(memory_space=pl.ANY)],
        out_specs=pl.BlockSpec(memory_space=pl.ANY),
        out_shape=jax.ShapeDtypeStruct((_TOTAL_ROWS, 4096), jnp.float32),
        scratch_shapes=[pltpu.SemaphoreType.DMA((_K,))],
    )(x2)
    return out.reshape(2, 4096, 4096)


# speedup vs baseline: 49.2688x; 49.0378x over previous
import jax
import jax.numpy as jnp
from jax.experimental import pallas as pl
from jax.experimental.pallas import tpu as pltpu

_TOTAL_ROWS = 2 * 4096
_SRC_OFFSET = 1 * 4096
_B = 512


def _copy_body(x_ref, out_ref):
    out_ref[...] = x_ref[...]


def kernel(x):
    x2 = x.reshape(4 * 4096, 4096)
    out = pl.pallas_call(
        _copy_body,
        grid=(_TOTAL_ROWS // _B,),
        in_specs=[
            pl.BlockSpec((_B, 4096), lambda i: (i + _SRC_OFFSET // _B, 0))
        ],
        out_specs=pl.BlockSpec((_B, 4096), lambda i: (i, 0)),
        out_shape=jax.ShapeDtypeStruct((_TOTAL_ROWS, 4096), jnp.float32),
    )(x2)
    return out.reshape(2, 4096, 4096)
